# baseline (speedup 1.0000x reference)
import jax
import jax.numpy as jnp
from jax import lax
from jax.experimental import pallas as pl
from jax.experimental.pallas import tpu as pltpu
from jax.experimental.pallas import tpu_sc as plsc

B, E, H, TOPK = 4096, 64, 256, 8
L = 16
NC, NS = 2, 16
NW = NC * NS
TPW = B // NW
EG = E // L

TB = 32
RB = TB * E

SZ_Q = TPW * E * 3
SZ_S = TPW * E
SZ_A = TPW * 3

_BF = jnp.bfloat16
_F32 = jnp.float32


def _dyng(x, idx):
    return jnp.take_along_axis(x, idx, axis=0, mode="promise_in_bounds")


NB = 4096


def _score_body(eit_ref, we_ref, be_ref, wse_ref, s1_ref):
    ee = lax.dot_general(we_ref[...].astype(_BF), eit_ref[...].astype(_BF),
                         (((1,), (0,)), ((), ())),
                         preferred_element_type=_F32) + be_ref[...]
    s1_ref[...] = lax.dot_general(wse_ref[...].astype(_BF), ee.astype(_BF),
                                  (((1,), (0,)), ((), ())),
                                  preferred_element_type=_F32)


_score = pl.pallas_call(
    _score_body,
    grid=(B * E // NB,),
    in_specs=[
        pl.BlockSpec((7, NB), lambda i: (0, i)),
        pl.BlockSpec((H, 7), lambda i: (0, 0)),
        pl.BlockSpec((H, 1), lambda i: (0, 0)),
        pl.BlockSpec((1, H), lambda i: (0, 0)),
    ],
    out_specs=pl.BlockSpec((1, NB), lambda i: (0, i)),
    out_shape=jax.ShapeDtypeStruct((1, B * E), _F32),
)


def _ctx_body(xct_ref, wc_ref, bc_ref, wsc_ref, bs_ref, cv_ref):
    ce = lax.dot_general(wc_ref[...].astype(_BF), xct_ref[...].astype(_BF),
                         (((1,), (1,)), ((), ())),
                         preferred_element_type=_F32) + bc_ref[...]
    cv_ref[...] = lax.dot_general(wsc_ref[...].astype(_BF), ce.astype(_BF),
                                  (((1,), (0,)), ((), ())),
                                  preferred_element_type=_F32) + bs_ref[...]


_ctx = pl.pallas_call(
    _ctx_body,
    out_shape=jax.ShapeDtypeStruct((1, B), _F32),
)


RPW = B * E // NW


def _deint_body(xq_h, xr_h, xk_h,
                eit_h,
                xq_v, xr_v, xk_v, f_v):
    wid = lax.axis_index("s") * NC + lax.axis_index("c")

    pltpu.sync_copy(xq_h.at[pl.ds(wid * (RPW * 3), RPW * 3)], xq_v)
    pltpu.sync_copy(xr_h.at[pl.ds(wid * (RPW * 2), RPW * 2)], xr_v)
    pltpu.sync_copy(xk_h.at[pl.ds(wid * (RPW * 2), RPW * 2)], xk_v)

    iota = lax.iota(jnp.int32, L)

    def feat(i, carry):
        r16 = i * L + iota
        for a in range(3):
            g = plsc.load_gather(xq_v, [r16 * 3 + a])
            plsc.store_scatter(f_v, [a * RPW + r16], g)
        for a in range(2):
            g = plsc.load_gather(xr_v, [r16 * 2 + a])
            plsc.store_scatter(f_v, [(3 + a) * RPW + r16], g)
            g = plsc.load_gather(xk_v, [r16 * 2 + a])
            plsc.store_scatter(f_v, [(5 + a) * RPW + r16], g)
        return carry

    lax.fori_loop(0, RPW // L, feat, 0)

    for a in range(7):
        pltpu.sync_copy(f_v.at[pl.ds(a * RPW, RPW)],
                        eit_h.at[pl.ds(a * (B * E) + wid * RPW, RPW)])


_deint = pl.kernel(
    _deint_body,
    out_type=jax.ShapeDtypeStruct((7 * B * E,), jnp.float32),
    mesh=plsc.VectorSubcoreMesh(core_axis_name="c", subcore_axis_name="s"),
    compiler_params=pltpu.CompilerParams(needs_layout_passes=False),
    scratch_types=[
        pltpu.VMEM((RPW * 3,), jnp.float32),
        pltpu.VMEM((RPW * 2,), jnp.float32),
        pltpu.VMEM((RPW * 2,), jnp.float32),
        pltpu.VMEM((RPW * 7,), jnp.float32),
    ],
)


NEG_INF = float("-inf")


def _sc_body(s1_h, cv_h, xq_h,
             act_h, gate_h, lg_h,
             s1_v, cv_v, xq_v, act_v, gate_v, lg_v):
    wid = lax.axis_index("s") * NC + lax.axis_index("c")

    pltpu.sync_copy(s1_h.at[pl.ds(wid * SZ_S, SZ_S)], s1_v)
    pltpu.sync_copy(cv_h.at[pl.ds(wid * TPW, TPW)], cv_v)
    pltpu.sync_copy(xq_h.at[pl.ds(wid * SZ_Q, SZ_Q)], xq_v)

    iota = lax.iota(jnp.int32, L)
    zeros16 = jnp.zeros((L,), jnp.float32)

    def zero_gate(i, carry):
        plsc.store_scatter(gate_v, [i * L + iota], zeros16)
        return carry

    lax.fori_loop(0, SZ_S // L, zero_gate, 0)

    def tile(j, carry):
        toks = j * L + iota
        cb = plsc.load_gather(cv_v, [toks])
        row = toks * E

        mA = [jnp.full((L,), NEG_INF, jnp.float32) for _ in range(8)]
        iA = [jnp.zeros((L,), jnp.int32) for _ in range(8)]
        mB = [jnp.full((L,), NEG_INF, jnp.float32) for _ in range(8)]
        iB = [jnp.zeros((L,), jnp.int32) for _ in range(8)]

        for e in range(E):
            v = plsc.load_gather(s1_v, [row + e]) + cb
            plsc.store_scatter(lg_v, [row + e], v)
            vi = jnp.full((L,), e, jnp.int32)
            m, mi = (mA, iA) if e < E // 2 else (mB, iB)
            for k in range(8):
                gt = v > m[k]
                nm = jnp.where(gt, v, m[k])
                ni = jnp.where(gt, vi, mi[k])
                v = jnp.where(gt, m[k], v)
                vi = jnp.where(gt, mi[k], vi)
                m[k], mi[k] = nm, ni

        sel_v, sel_i = [], []
        for k in range(8):
            keep = mA[k] >= mB[7 - k]
            sel_v.append(jnp.where(keep, mA[k], mB[7 - k]))
            sel_i.append(jnp.where(keep, iA[k], iB[7 - k]))

        mx = jnp.maximum(mA[0], mB[0])
        ex = [jnp.exp(sv - mx) for sv in sel_v]
        z = ex[0]
        for k in range(1, 8):
            z = z + ex[k]
        rz = 1.0 / z
        acc0 = acc1 = acc2 = zeros16
        qrow = toks * (E * 3)
        for k in range(8):
            g = ex[k] * rz
            plsc.store_scatter(gate_v, [row + sel_i[k]], g)
            qb = qrow + sel_i[k] * 3
            acc0 = acc0 + g * plsc.load_gather(xq_v, [qb])
            acc1 = acc1 + g * plsc.load_gather(xq_v, [qb + 1])
            acc2 = acc2 + g * plsc.load_gather(xq_v, [qb + 2])

        arow = toks * 3
        plsc.store_scatter(act_v, [arow], acc0)
        plsc.store_scatter(act_v, [arow + 1], acc1)
        plsc.store_scatter(act_v, [arow + 2], acc2)
        return carry

    lax.fori_loop(0, TPW // L, tile, 0)

    pltpu.sync_copy(act_v, act_h.at[pl.ds(wid * SZ_A, SZ_A)])
    pltpu.sync_copy(gate_v, gate_h.at[pl.ds(wid * SZ_S, SZ_S)])
    pltpu.sync_copy(lg_v, lg_h.at[pl.ds(wid * SZ_S, SZ_S)])


_sc_router = pl.kernel(
    _sc_body,
    out_type=[
        jax.ShapeDtypeStruct((B * 3,), jnp.float32),
        jax.ShapeDtypeStruct((B * E,), jnp.float32),
        jax.ShapeDtypeStruct((B * E,), jnp.float32),
    ],
    mesh=plsc.VectorSubcoreMesh(core_axis_name="c", subcore_axis_name="s"),
    compiler_params=pltpu.CompilerParams(needs_layout_passes=False),
    scratch_types=[
        pltpu.VMEM((SZ_S,), jnp.float32),
        pltpu.VMEM((TPW,), jnp.float32),
        pltpu.VMEM((SZ_Q,), jnp.float32),
        pltpu.VMEM((SZ_A,), jnp.float32),
        pltpu.VMEM((SZ_S,), jnp.float32),
        pltpu.VMEM((SZ_S,), jnp.float32),
    ],
)


def kernel(x_context, x_q_values, x_reward, x_risk, Wc, bc, We, be, Ws, bs):
    assert x_q_values.shape == (B, E, 3) and x_context.shape == (B, 68)
    eit = _deint(x_q_values.reshape(-1), x_reward.reshape(-1),
                 x_risk.reshape(-1))
    s1 = _score(eit.reshape(7, B * E), We, be.reshape(H, 1), Ws[:, :H])
    cv = _ctx(x_context, Wc, bc.reshape(H, 1), Ws[:, H:],
              bs.reshape(1, 1))
    act, gate, lg = _sc_router(s1.reshape(-1), cv.reshape(-1),
                               x_q_values.reshape(-1))
    return act.reshape(B, 3), gate.reshape(B, E), lg.reshape(B, E)


# speedup vs baseline: 4.6084x; 4.6084x over previous
import jax
import jax.numpy as jnp
from jax import lax
from jax.experimental import pallas as pl
from jax.experimental.pallas import tpu as pltpu
from jax.experimental.pallas import tpu_sc as plsc

B, E, H, TOPK = 4096, 64, 256, 8
L = 16
NC, NS = 2, 16
NW = NC * NS
TPW = B // NW
EG = E // L

TB = 32
RB = TB * E

SZ_Q = TPW * E * 3
SZ_S = TPW * E
SZ_A = TPW * 3

_BF = jnp.bfloat16
_F32 = jnp.float32


def _dyng(x, idx):
    return jnp.take_along_axis(x, idx, axis=0, mode="promise_in_bounds")


NB = 4096


def _score_body(eit_ref, we_ref, be_ref, wse_ref, s1_ref):
    ee = lax.dot_general(we_ref[...].astype(_BF), eit_ref[...].astype(_BF),
                         (((1,), (0,)), ((), ())),
                         preferred_element_type=_F32) + be_ref[...]
    s1_ref[...] = lax.dot_general(wse_ref[...].astype(_BF), ee.astype(_BF),
                                  (((1,), (0,)), ((), ())),
                                  preferred_element_type=_F32)


_score = pl.pallas_call(
    _score_body,
    grid=(B * E // NB,),
    in_specs=[
        pl.BlockSpec((7, NB), lambda i: (0, i)),
        pl.BlockSpec((H, 7), lambda i: (0, 0)),
        pl.BlockSpec((H, 1), lambda i: (0, 0)),
        pl.BlockSpec((1, H), lambda i: (0, 0)),
    ],
    out_specs=pl.BlockSpec((1, NB), lambda i: (0, i)),
    out_shape=jax.ShapeDtypeStruct((1, B * E), _F32),
)


def _ctx_body(xct_ref, wc_ref, bc_ref, wsc_ref, bs_ref, cv_ref):
    ce = lax.dot_general(wc_ref[...].astype(_BF), xct_ref[...].astype(_BF),
                         (((1,), (0,)), ((), ())),
                         preferred_element_type=_F32) + bc_ref[...]
    cv_ref[...] = lax.dot_general(wsc_ref[...].astype(_BF), ce.astype(_BF),
                                  (((1,), (0,)), ((), ())),
                                  preferred_element_type=_F32) + bs_ref[...]


_ctx = pl.pallas_call(
    _ctx_body,
    out_shape=jax.ShapeDtypeStruct((1, B), _F32),
)


NEG_INF = float("-inf")


def _sc_body(s3_h, cv_h, q3_h,
             act_h, gate_h, lg_h,
             s1_v, cv_v, q_v, act_v, gate_v, lg_v):
    wid = lax.axis_index("s") * NC + lax.axis_index("c")

    pltpu.sync_copy(s3_h.at[:, pl.ds(wid * TPW, TPW)], s1_v)
    pltpu.sync_copy(cv_h.at[pl.ds(wid * TPW, TPW)], cv_v)
    for a in range(3):
        pltpu.sync_copy(q3_h.at[a, :, pl.ds(wid * TPW, TPW)], q_v.at[a])

    iota = lax.iota(jnp.int32, L)
    zeros16 = jnp.zeros((L,), jnp.float32)

    def zero_gate(i, carry):
        plsc.store_scatter(gate_v, [i * L + iota], zeros16)
        return carry

    lax.fori_loop(0, SZ_S // L, zero_gate, 0)

    def tile(j, carry):
        toks = j * L + iota
        cb = plsc.load_gather(cv_v, [toks])
        row = toks * E

        mA = [jnp.full((L,), NEG_INF, jnp.float32) for _ in range(8)]
        iA = [jnp.zeros((L,), jnp.int32) for _ in range(8)]
        mB = [jnp.full((L,), NEG_INF, jnp.float32) for _ in range(8)]
        iB = [jnp.zeros((L,), jnp.int32) for _ in range(8)]

        for e in range(E):
            v = plsc.load_gather(s1_v, [jnp.full((L,), e, jnp.int32), toks]) + cb
            plsc.store_scatter(lg_v, [row + e], v)
            vi = jnp.full((L,), e, jnp.int32)
            m, mi = (mA, iA) if e < E // 2 else (mB, iB)
            for k in range(8):
                gt = v > m[k]
                nm = jnp.where(gt, v, m[k])
                ni = jnp.where(gt, vi, mi[k])
                v = jnp.where(gt, m[k], v)
                vi = jnp.where(gt, mi[k], vi)
                m[k], mi[k] = nm, ni

        sel_v, sel_i = [], []
        for k in range(8):
            keep = mA[k] >= mB[7 - k]
            sel_v.append(jnp.where(keep, mA[k], mB[7 - k]))
            sel_i.append(jnp.where(keep, iA[k], iB[7 - k]))

        mx = jnp.maximum(mA[0], mB[0])
        ex = [jnp.exp(sv - mx) for sv in sel_v]
        z = ex[0]
        for k in range(1, 8):
            z = z + ex[k]
        rz = 1.0 / z
        acc0 = acc1 = acc2 = zeros16
        for k in range(8):
            g = ex[k] * rz
            plsc.store_scatter(gate_v, [row + sel_i[k]], g)
            acc0 = acc0 + g * plsc.load_gather(
                q_v, [jnp.zeros((L,), jnp.int32), sel_i[k], toks])
            acc1 = acc1 + g * plsc.load_gather(
                q_v, [jnp.full((L,), 1, jnp.int32), sel_i[k], toks])
            acc2 = acc2 + g * plsc.load_gather(
                q_v, [jnp.full((L,), 2, jnp.int32), sel_i[k], toks])

        arow = toks * 3
        plsc.store_scatter(act_v, [arow], acc0)
        plsc.store_scatter(act_v, [arow + 1], acc1)
        plsc.store_scatter(act_v, [arow + 2], acc2)
        return carry

    lax.fori_loop(0, TPW // L, tile, 0)

    pltpu.sync_copy(act_v, act_h.at[pl.ds(wid * SZ_A, SZ_A)])
    pltpu.sync_copy(gate_v, gate_h.at[pl.ds(wid * SZ_S, SZ_S)])
    pltpu.sync_copy(lg_v, lg_h.at[pl.ds(wid * SZ_S, SZ_S)])


_sc_router = pl.kernel(
    _sc_body,
    out_type=[
        jax.ShapeDtypeStruct((B * 3,), jnp.float32),
        jax.ShapeDtypeStruct((B * E,), jnp.float32),
        jax.ShapeDtypeStruct((B * E,), jnp.float32),
    ],
    mesh=plsc.VectorSubcoreMesh(core_axis_name="c", subcore_axis_name="s"),
    compiler_params=pltpu.CompilerParams(needs_layout_passes=False),
    scratch_types=[
        pltpu.VMEM((E, TPW), jnp.float32),
        pltpu.VMEM((TPW,), jnp.float32),
        pltpu.VMEM((3, E, TPW), jnp.float32),
        pltpu.VMEM((SZ_A,), jnp.float32),
        pltpu.VMEM((SZ_S,), jnp.float32),
        pltpu.VMEM((SZ_S,), jnp.float32),
    ],
)


def kernel(x_context, x_q_values, x_reward, x_risk, Wc, bc, We, be, Ws, bs):
    assert x_q_values.shape == (B, E, 3) and x_context.shape == (B, 68)
    eit = jnp.concatenate(
        [jnp.transpose(x_q_values, (2, 1, 0)).reshape(3, B * E),
         jnp.transpose(x_reward, (2, 1, 0)).reshape(2, B * E),
         jnp.transpose(x_risk, (2, 1, 0)).reshape(2, B * E)], axis=0)
    s1 = _score(eit, We, be.reshape(H, 1), Ws[:, :H])
    cv = _ctx(x_context.T, Wc, bc.reshape(H, 1), Ws[:, H:], bs.reshape(1, 1))
    act, gate, lg = _sc_router(s1.reshape(E, B), cv.reshape(-1),
                               eit[:3].reshape(3, E, B))
    return act.reshape(B, 3), gate.reshape(B, E), lg.reshape(B, E)
